# baseline (speedup 1.0000x reference)
import functools

import jax
import jax.numpy as jnp
from jax.experimental import pallas as pl
from jax.experimental.pallas import tpu as pltpu

B, T = 4096, 50
D_IN, ENC, HID = 11, 128, 128
STEPS = 50
BB = 256
GX_CHUNK = 10


def _fused_kernel(x_ref, w1_ref, b1_ref, w2_ref, b2_ref,
                  wih_ref, whh_ref, bc_ref,
                  cw1_ref, cb1_ref, cw2_ref, cb2_ref,
                  lw1_ref, lb1_ref, lw2_ref, lb2_ref,
                  coords_ref, labels_ref, gx_scr):
    f32 = jnp.float32
    bf16 = jnp.bfloat16
    x = x_ref[...].reshape(T * BB, D_IN).astype(bf16)
    z = jnp.maximum(jnp.dot(x, w1_ref[...].astype(bf16),
                            preferred_element_type=f32) + b1_ref[...], 0.0)
    z = jnp.maximum(jnp.dot(z.astype(bf16), w2_ref[...].astype(bf16),
                            preferred_element_type=f32) + b2_ref[...], 0.0)
    zb = z.astype(bf16)

    wih = wih_ref[...].astype(bf16)
    bc = bc_ref[...]
    for c in range(T // GX_CHUNK):
        rows = zb[c * GX_CHUNK * BB:(c + 1) * GX_CHUNK * BB]
        gx = jnp.dot(rows, wih, preferred_element_type=f32) + bc
        gx_scr[c * GX_CHUNK:(c + 1) * GX_CHUNK] = gx.reshape(GX_CHUNK, BB, 4 * HID)

    whh = whh_ref[...].astype(bf16)

    def step(t, carry):
        h, c = carry
        gates = gx_scr[t] + jnp.dot(h.astype(bf16), whh,
                                    preferred_element_type=f32)
        i_t = jax.nn.sigmoid(gates[:, 0 * HID:1 * HID])
        f_t = jax.nn.sigmoid(gates[:, 1 * HID:2 * HID])
        g_t = jnp.tanh(gates[:, 2 * HID:3 * HID])
        o_t = jax.nn.sigmoid(gates[:, 3 * HID:4 * HID])
        c_new = f_t * c + i_t * g_t
        h_new = o_t * jnp.tanh(c_new)
        return h_new, c_new

    h0 = jnp.zeros((BB, HID), dtype=f32)
    c0 = jnp.zeros((BB, HID), dtype=f32)
    h, _ = jax.lax.fori_loop(0, T, step, (h0, c0))

    hc = jnp.maximum(jnp.dot(h, cw1_ref[...], preferred_element_type=f32)
                     + cb1_ref[...], 0.0)
    coords_ref[...] = jnp.dot(hc, cw2_ref[...], preferred_element_type=f32) + cb2_ref[...]
    hl = jnp.maximum(jnp.dot(h, lw1_ref[...], preferred_element_type=f32)
                     + lb1_ref[...], 0.0)
    labels_ref[...] = jnp.dot(hl, lw2_ref[...], preferred_element_type=f32) + lb2_ref[...]


def _full(shape):
    return pl.BlockSpec(shape, lambda i: (0,) * len(shape))


@functools.partial(jax.jit, static_argnames=("interpret",))
def _run(x, enc_W1, enc_b1, enc_W2, enc_b2, W_ih, W_hh, b_cat,
         coord_W1, coord_b1, coord_W2, coord_b2,
         lab_W1, lab_b1, lab_W2, lab_b2, interpret=False):
    n_tiles = B // BB
    out_shapes = (
        jax.ShapeDtypeStruct((B, 3 * STEPS), jnp.float32),
        jax.ShapeDtypeStruct((B, STEPS), jnp.float32),
    )
    return pl.pallas_call(
        _fused_kernel,
        grid=(n_tiles,),
        in_specs=[
            pl.BlockSpec((T, BB, D_IN), lambda i: (0, i, 0)),
            _full((D_IN, ENC)), _full((1, ENC)),
            _full((ENC, ENC)), _full((1, ENC)),
            _full((ENC, 4 * HID)), _full((HID, 4 * HID)), _full((1, 4 * HID)),
            _full((HID, HID)), _full((1, HID)),
            _full((HID, 3 * STEPS)), _full((1, 3 * STEPS)),
            _full((HID, HID // 2)), _full((1, HID // 2)),
            _full((HID // 2, STEPS)), _full((1, STEPS)),
        ],
        out_specs=(
            pl.BlockSpec((BB, 3 * STEPS), lambda i: (i, 0)),
            pl.BlockSpec((BB, STEPS), lambda i: (i, 0)),
        ),
        out_shape=out_shapes,
        scratch_shapes=[pltpu.VMEM((T, BB, 4 * HID), jnp.float32)],
        compiler_params=pltpu.CompilerParams(
            dimension_semantics=("parallel",),
        ),
        interpret=interpret,
    )(x, enc_W1, enc_b1, enc_W2, enc_b2, W_ih, W_hh, b_cat,
      coord_W1, coord_b1, coord_W2, coord_b2,
      lab_W1, lab_b1, lab_W2, lab_b2)


def kernel(obs_l, obs_c, obs_m, enc_W1, enc_b1, enc_W2, enc_b2,
           W_ih, W_hh, b_ih, b_hh,
           coord_W1, coord_b1, coord_W2, coord_b2,
           lab_W1, lab_b1, lab_W2, lab_b2):
    x = jnp.concatenate([obs_l, obs_c, obs_m], axis=-1)
    x = jnp.swapaxes(x, 0, 1)
    b_cat = (b_ih + b_hh).reshape(1, 4 * HID)
    return _run(x, enc_W1, enc_b1.reshape(1, ENC), enc_W2, enc_b2.reshape(1, ENC),
                W_ih, W_hh, b_cat,
                coord_W1, coord_b1.reshape(1, HID), coord_W2, coord_b2.reshape(1, 3 * STEPS),
                lab_W1, lab_b1.reshape(1, HID // 2), lab_W2, lab_b2.reshape(1, STEPS))


# speedup vs baseline: 1.1170x; 1.1170x over previous
import functools

import jax
import jax.numpy as jnp
from jax.experimental import pallas as pl
from jax.experimental.pallas import tpu as pltpu

B, T = 4096, 50
D_IN, ENC, HID = 11, 128, 128
STEPS = 50
BB = 512
HB = BB // 2
GX_CHUNK = 5


def _fused_kernel(x_ref, w1_ref, b1_ref, w2_ref, b2_ref,
                  wih_ref, whh_ref, bc_ref,
                  cw1_ref, cb1_ref, cw2_ref, cb2_ref,
                  lw1_ref, lb1_ref, lw2_ref, lb2_ref,
                  coords_ref, labels_ref, gx_scr):
    f32 = jnp.float32
    bf16 = jnp.bfloat16
    w1 = w1_ref[...].astype(bf16)
    b1 = b1_ref[...]
    w2 = w2_ref[...].astype(bf16)
    b2 = b2_ref[...]
    wih = wih_ref[...].astype(bf16)
    bc = bc_ref[...]
    for c in range(T // GX_CHUNK):
        xc = x_ref[c * GX_CHUNK:(c + 1) * GX_CHUNK].reshape(GX_CHUNK * BB, D_IN).astype(bf16)
        zc = jnp.maximum(jnp.dot(xc, w1, preferred_element_type=f32) + b1, 0.0)
        zc = jnp.maximum(jnp.dot(zc.astype(bf16), w2, preferred_element_type=f32) + b2, 0.0)
        gx = (jnp.dot(zc.astype(bf16), wih, preferred_element_type=f32) + bc).astype(bf16)
        gx_scr[c * GX_CHUNK:(c + 1) * GX_CHUNK] = gx.reshape(GX_CHUNK, BB, 4 * HID)

    whh = whh_ref[...].astype(bf16)

    def act(gates, c):
        i_t = jax.nn.sigmoid(gates[:, 0 * HID:1 * HID])
        f_t = jax.nn.sigmoid(gates[:, 1 * HID:2 * HID])
        g_t = jnp.tanh(gates[:, 2 * HID:3 * HID])
        o_t = jax.nn.sigmoid(gates[:, 3 * HID:4 * HID])
        c_new = f_t * c + i_t * g_t
        h_new = o_t * jnp.tanh(c_new)
        return h_new, c_new

    def step(t, carry):
        ha, ca, hb, cb = carry
        gx = gx_scr[t].astype(f32)
        ga = gx[0:HB] + jnp.dot(ha.astype(bf16), whh, preferred_element_type=f32)
        gb = gx[HB:BB] + jnp.dot(hb.astype(bf16), whh, preferred_element_type=f32)
        ha, ca = act(ga, ca)
        hb, cb = act(gb, cb)
        return ha, ca, hb, cb

    zero = jnp.zeros((HB, HID), dtype=f32)
    ha, _, hb, _ = jax.lax.fori_loop(0, T, step, (zero, zero, zero, zero))
    h = jnp.concatenate([ha, hb], axis=0)

    hc = jnp.maximum(jnp.dot(h, cw1_ref[...], preferred_element_type=f32)
                     + cb1_ref[...], 0.0)
    coords_ref[...] = jnp.dot(hc, cw2_ref[...], preferred_element_type=f32) + cb2_ref[...]
    hl = jnp.maximum(jnp.dot(h, lw1_ref[...], preferred_element_type=f32)
                     + lb1_ref[...], 0.0)
    labels_ref[...] = jnp.dot(hl, lw2_ref[...], preferred_element_type=f32) + lb2_ref[...]


def _full(shape):
    return pl.BlockSpec(shape, lambda i: (0,) * len(shape))


@functools.partial(jax.jit, static_argnames=("interpret",))
def _run(x, enc_W1, enc_b1, enc_W2, enc_b2, W_ih, W_hh, b_cat,
         coord_W1, coord_b1, coord_W2, coord_b2,
         lab_W1, lab_b1, lab_W2, lab_b2, interpret=False):
    n_tiles = B // BB
    out_shapes = (
        jax.ShapeDtypeStruct((B, 3 * STEPS), jnp.float32),
        jax.ShapeDtypeStruct((B, STEPS), jnp.float32),
    )
    return pl.pallas_call(
        _fused_kernel,
        grid=(n_tiles,),
        in_specs=[
            pl.BlockSpec((T, BB, D_IN), lambda i: (0, i, 0)),
            _full((D_IN, ENC)), _full((1, ENC)),
            _full((ENC, ENC)), _full((1, ENC)),
            _full((ENC, 4 * HID)), _full((HID, 4 * HID)), _full((1, 4 * HID)),
            _full((HID, HID)), _full((1, HID)),
            _full((HID, 3 * STEPS)), _full((1, 3 * STEPS)),
            _full((HID, HID // 2)), _full((1, HID // 2)),
            _full((HID // 2, STEPS)), _full((1, STEPS)),
        ],
        out_specs=(
            pl.BlockSpec((BB, 3 * STEPS), lambda i: (i, 0)),
            pl.BlockSpec((BB, STEPS), lambda i: (i, 0)),
        ),
        out_shape=out_shapes,
        scratch_shapes=[pltpu.VMEM((T, BB, 4 * HID), jnp.bfloat16)],
        compiler_params=pltpu.CompilerParams(
            dimension_semantics=("parallel",),
        ),
        interpret=interpret,
    )(x, enc_W1, enc_b1, enc_W2, enc_b2, W_ih, W_hh, b_cat,
      coord_W1, coord_b1, coord_W2, coord_b2,
      lab_W1, lab_b1, lab_W2, lab_b2)


def kernel(obs_l, obs_c, obs_m, enc_W1, enc_b1, enc_W2, enc_b2,
           W_ih, W_hh, b_ih, b_hh,
           coord_W1, coord_b1, coord_W2, coord_b2,
           lab_W1, lab_b1, lab_W2, lab_b2):
    x = jnp.concatenate([obs_l, obs_c, obs_m], axis=-1)
    x = jnp.swapaxes(x, 0, 1)
    b_cat = (b_ih + b_hh).reshape(1, 4 * HID)
    return _run(x, enc_W1, enc_b1.reshape(1, ENC), enc_W2, enc_b2.reshape(1, ENC),
                W_ih, W_hh, b_cat,
                coord_W1, coord_b1.reshape(1, HID), coord_W2, coord_b2.reshape(1, 3 * STEPS),
                lab_W1, lab_b1.reshape(1, HID // 2), lab_W2, lab_b2.reshape(1, STEPS))


# speedup vs baseline: 1.3613x; 1.2187x over previous
import functools

import jax
import jax.numpy as jnp
from jax.experimental import pallas as pl
from jax.experimental.pallas import tpu as pltpu

B, T = 4096, 50
D_IN, ENC, HID = 11, 128, 128
STEPS = 50
BB = 512
HB = BB // 2
GX_CHUNK = 5


def _fused_kernel(x_ref, w1_ref, b1_ref, w2_ref, b2_ref,
                  wih_ref, whh_ref, bc_ref,
                  cw1_ref, cb1_ref, cw2_ref, cb2_ref,
                  lw1_ref, lb1_ref, lw2_ref, lb2_ref,
                  coords_ref, labels_ref, gx_scr):
    f32 = jnp.float32
    bf16 = jnp.bfloat16
    w1 = w1_ref[...].astype(bf16)
    b1 = b1_ref[...]
    w2 = w2_ref[...].astype(bf16)
    b2 = b2_ref[...]
    wih = wih_ref[...].astype(bf16)
    bc = bc_ref[...]
    for c in range(T // GX_CHUNK):
        xc = x_ref[c * GX_CHUNK:(c + 1) * GX_CHUNK].reshape(GX_CHUNK * BB, D_IN).astype(bf16)
        zc = jnp.maximum(jnp.dot(xc, w1, preferred_element_type=f32) + b1, 0.0)
        zc = jnp.maximum(jnp.dot(zc.astype(bf16), w2, preferred_element_type=f32) + b2, 0.0)
        gx = (jnp.dot(zc.astype(bf16), wih, preferred_element_type=f32) + bc).astype(bf16)
        gx_scr[c * GX_CHUNK:(c + 1) * GX_CHUNK] = gx.reshape(GX_CHUNK, BB, 4 * HID)

    whh = whh_ref[...].astype(bf16)

    def act(gates, c):
        i_t = jax.nn.sigmoid(gates[:, 0 * HID:1 * HID])
        f_t = jax.nn.sigmoid(gates[:, 1 * HID:2 * HID])
        g_t = jnp.tanh(gates[:, 2 * HID:3 * HID])
        o_t = jax.nn.sigmoid(gates[:, 3 * HID:4 * HID])
        c_new = f_t * c + i_t * g_t
        h_new = o_t * jnp.tanh(c_new)
        return h_new, c_new

    zero = jnp.zeros((HB, HID), dtype=f32)
    ha, ca, hb, cb = zero, zero, zero, zero
    for t in range(T):
        gx = gx_scr[t].astype(f32)
        ga = gx[0:HB] + jnp.dot(ha.astype(bf16), whh, preferred_element_type=f32)
        gb = gx[HB:BB] + jnp.dot(hb.astype(bf16), whh, preferred_element_type=f32)
        ha, ca = act(ga, ca)
        hb, cb = act(gb, cb)
    h = jnp.concatenate([ha, hb], axis=0)

    hc = jnp.maximum(jnp.dot(h, cw1_ref[...], preferred_element_type=f32)
                     + cb1_ref[...], 0.0)
    coords_ref[...] = jnp.dot(hc, cw2_ref[...], preferred_element_type=f32) + cb2_ref[...]
    hl = jnp.maximum(jnp.dot(h, lw1_ref[...], preferred_element_type=f32)
                     + lb1_ref[...], 0.0)
    labels_ref[...] = jnp.dot(hl, lw2_ref[...], preferred_element_type=f32) + lb2_ref[...]


def _full(shape):
    return pl.BlockSpec(shape, lambda i: (0,) * len(shape))


@functools.partial(jax.jit, static_argnames=("interpret",))
def _run(x, enc_W1, enc_b1, enc_W2, enc_b2, W_ih, W_hh, b_cat,
         coord_W1, coord_b1, coord_W2, coord_b2,
         lab_W1, lab_b1, lab_W2, lab_b2, interpret=False):
    n_tiles = B // BB
    out_shapes = (
        jax.ShapeDtypeStruct((B, 3 * STEPS), jnp.float32),
        jax.ShapeDtypeStruct((B, STEPS), jnp.float32),
    )
    return pl.pallas_call(
        _fused_kernel,
        grid=(n_tiles,),
        in_specs=[
            pl.BlockSpec((T, BB, D_IN), lambda i: (0, i, 0)),
            _full((D_IN, ENC)), _full((1, ENC)),
            _full((ENC, ENC)), _full((1, ENC)),
            _full((ENC, 4 * HID)), _full((HID, 4 * HID)), _full((1, 4 * HID)),
            _full((HID, HID)), _full((1, HID)),
            _full((HID, 3 * STEPS)), _full((1, 3 * STEPS)),
            _full((HID, HID // 2)), _full((1, HID // 2)),
            _full((HID // 2, STEPS)), _full((1, STEPS)),
        ],
        out_specs=(
            pl.BlockSpec((BB, 3 * STEPS), lambda i: (i, 0)),
            pl.BlockSpec((BB, STEPS), lambda i: (i, 0)),
        ),
        out_shape=out_shapes,
        scratch_shapes=[pltpu.VMEM((T, BB, 4 * HID), jnp.bfloat16)],
        compiler_params=pltpu.CompilerParams(
            dimension_semantics=("parallel",),
        ),
        interpret=interpret,
    )(x, enc_W1, enc_b1, enc_W2, enc_b2, W_ih, W_hh, b_cat,
      coord_W1, coord_b1, coord_W2, coord_b2,
      lab_W1, lab_b1, lab_W2, lab_b2)


def kernel(obs_l, obs_c, obs_m, enc_W1, enc_b1, enc_W2, enc_b2,
           W_ih, W_hh, b_ih, b_hh,
           coord_W1, coord_b1, coord_W2, coord_b2,
           lab_W1, lab_b1, lab_W2, lab_b2):
    x = jnp.concatenate([obs_l, obs_c, obs_m], axis=-1)
    x = jnp.swapaxes(x, 0, 1)
    b_cat = (b_ih + b_hh).reshape(1, 4 * HID)
    return _run(x, enc_W1, enc_b1.reshape(1, ENC), enc_W2, enc_b2.reshape(1, ENC),
                W_ih, W_hh, b_cat,
                coord_W1, coord_b1.reshape(1, HID), coord_W2, coord_b2.reshape(1, 3 * STEPS),
                lab_W1, lab_b1.reshape(1, HID // 2), lab_W2, lab_b2.reshape(1, STEPS))


# speedup vs baseline: 2.4586x; 1.8060x over previous
import functools

import jax
import jax.numpy as jnp
from jax.experimental import pallas as pl
from jax.experimental.pallas import tpu as pltpu

B, T = 4096, 50
D_IN, ENC, HID = 11, 128, 128
STEPS = 50
BB = 512
HB = BB // 2
GX_CHUNK = 5


def _fused_kernel(x_ref, w1_ref, b1_ref, w2_ref, b2_ref,
                  wih_ref, whh_ref, bc_ref,
                  cw1_ref, cb1_ref, cw2_ref, cb2_ref,
                  lw1_ref, lb1_ref, lw2_ref, lb2_ref,
                  coords_ref, labels_ref, gx_scr):
    f32 = jnp.float32
    bf16 = jnp.bfloat16
    w1 = w1_ref[...].astype(bf16)
    b1 = b1_ref[...]
    w2 = w2_ref[...].astype(bf16)
    b2 = b2_ref[...]
    wih = wih_ref[...].astype(bf16)
    bc = bc_ref[...]
    for c in range(T // GX_CHUNK):
        xc = x_ref[c * GX_CHUNK:(c + 1) * GX_CHUNK].reshape(GX_CHUNK * BB, D_IN)
        zc = jnp.maximum(jnp.dot(xc, w1, preferred_element_type=f32) + b1, 0.0)
        zc = jnp.maximum(jnp.dot(zc.astype(bf16), w2, preferred_element_type=f32) + b2, 0.0)
        gx = (jnp.dot(zc.astype(bf16), wih, preferred_element_type=f32) + bc).astype(bf16)
        gx_scr[c * GX_CHUNK:(c + 1) * GX_CHUNK] = gx.reshape(GX_CHUNK, BB, 4 * HID)

    whh = whh_ref[...].astype(bf16)

    def act(gates, c):
        i_t = jax.nn.sigmoid(gates[:, 0 * HID:1 * HID])
        f_t = jax.nn.sigmoid(gates[:, 1 * HID:2 * HID])
        g_t = jnp.tanh(gates[:, 2 * HID:3 * HID])
        o_t = jax.nn.sigmoid(gates[:, 3 * HID:4 * HID])
        c_new = f_t * c + i_t * g_t
        h_new = o_t * jnp.tanh(c_new)
        return h_new, c_new

    zero = jnp.zeros((HB, HID), dtype=f32)
    ha, ca, hb, cb = zero, zero, zero, zero
    for t in range(T):
        gx = gx_scr[t].astype(f32)
        ga = gx[0:HB] + jnp.dot(ha.astype(bf16), whh, preferred_element_type=f32)
        gb = gx[HB:BB] + jnp.dot(hb.astype(bf16), whh, preferred_element_type=f32)
        ha, ca = act(ga, ca)
        hb, cb = act(gb, cb)
    h = jnp.concatenate([ha, hb], axis=0)

    hc = jnp.maximum(jnp.dot(h, cw1_ref[...], preferred_element_type=f32)
                     + cb1_ref[...], 0.0)
    coords_ref[...] = jnp.dot(hc, cw2_ref[...], preferred_element_type=f32) + cb2_ref[...]
    hl = jnp.maximum(jnp.dot(h, lw1_ref[...], preferred_element_type=f32)
                     + lb1_ref[...], 0.0)
    labels_ref[...] = jnp.dot(hl, lw2_ref[...], preferred_element_type=f32) + lb2_ref[...]


def _full(shape):
    return pl.BlockSpec(shape, lambda i: (0,) * len(shape))


@functools.partial(jax.jit, static_argnames=("interpret",))
def _run(x, enc_W1, enc_b1, enc_W2, enc_b2, W_ih, W_hh, b_cat,
         coord_W1, coord_b1, coord_W2, coord_b2,
         lab_W1, lab_b1, lab_W2, lab_b2, interpret=False):
    n_tiles = B // BB
    out_shapes = (
        jax.ShapeDtypeStruct((B, 3 * STEPS), jnp.float32),
        jax.ShapeDtypeStruct((B, STEPS), jnp.float32),
    )
    return pl.pallas_call(
        _fused_kernel,
        grid=(n_tiles,),
        in_specs=[
            pl.BlockSpec((T, BB, D_IN), lambda i: (0, i, 0)),
            _full((D_IN, ENC)), _full((1, ENC)),
            _full((ENC, ENC)), _full((1, ENC)),
            _full((ENC, 4 * HID)), _full((HID, 4 * HID)), _full((1, 4 * HID)),
            _full((HID, HID)), _full((1, HID)),
            _full((HID, 3 * STEPS)), _full((1, 3 * STEPS)),
            _full((HID, HID // 2)), _full((1, HID // 2)),
            _full((HID // 2, STEPS)), _full((1, STEPS)),
        ],
        out_specs=(
            pl.BlockSpec((BB, 3 * STEPS), lambda i: (i, 0)),
            pl.BlockSpec((BB, STEPS), lambda i: (i, 0)),
        ),
        out_shape=out_shapes,
        scratch_shapes=[pltpu.VMEM((T, BB, 4 * HID), jnp.bfloat16)],
        compiler_params=pltpu.CompilerParams(
            dimension_semantics=("parallel",),
        ),
        interpret=interpret,
    )(x, enc_W1, enc_b1, enc_W2, enc_b2, W_ih, W_hh, b_cat,
      coord_W1, coord_b1, coord_W2, coord_b2,
      lab_W1, lab_b1, lab_W2, lab_b2)


def kernel(obs_l, obs_c, obs_m, enc_W1, enc_b1, enc_W2, enc_b2,
           W_ih, W_hh, b_ih, b_hh,
           coord_W1, coord_b1, coord_W2, coord_b2,
           lab_W1, lab_b1, lab_W2, lab_b2):
    x = jnp.concatenate([obs_l, obs_c, obs_m], axis=-1)
    x = jnp.swapaxes(x, 0, 1).astype(jnp.bfloat16)
    b_cat = (b_ih + b_hh).reshape(1, 4 * HID)
    return _run(x, enc_W1, enc_b1.reshape(1, ENC), enc_W2, enc_b2.reshape(1, ENC),
                W_ih, W_hh, b_cat,
                coord_W1, coord_b1.reshape(1, HID), coord_W2, coord_b2.reshape(1, 3 * STEPS),
                lab_W1, lab_b1.reshape(1, HID // 2), lab_W2, lab_b2.reshape(1, STEPS))


# speedup vs baseline: 2.4664x; 1.0032x over previous
import functools

import jax
import jax.numpy as jnp
from jax.experimental import pallas as pl
from jax.experimental.pallas import tpu as pltpu

B, T = 4096, 50
D_IN, ENC, HID = 11, 128, 128
STEPS = 50
BB = 512
HB = BB // 2
GX_CHUNK = 5
D_INA = D_IN + 1


def _fused_kernel(x_ref, w1a_ref, w2a_ref, wih_ref, whha_ref,
                  cw1_ref, cb1_ref, cw2_ref, cb2_ref,
                  lw1_ref, lb1_ref, lw2_ref, lb2_ref,
                  coords_ref, labels_ref, gx_scr):
    f32 = jnp.float32
    bf16 = jnp.bfloat16
    w1a = w1a_ref[...].astype(bf16)
    w2a = w2a_ref[...].astype(bf16)
    wih = wih_ref[...].astype(bf16)
    ones_chunk = jnp.ones((GX_CHUNK * BB, 1), dtype=bf16)
    for c in range(T // GX_CHUNK):
        xc = x_ref[c * GX_CHUNK:(c + 1) * GX_CHUNK].reshape(GX_CHUNK * BB, D_INA)
        z1 = jnp.maximum(jnp.dot(xc, w1a, preferred_element_type=f32).astype(bf16), 0.0)
        z1 = jnp.concatenate([z1, ones_chunk], axis=1)
        z2 = jnp.maximum(jnp.dot(z1, w2a, preferred_element_type=f32).astype(bf16), 0.0)
        gx = jnp.dot(z2, wih, preferred_element_type=f32).astype(bf16)
        gx_scr[c * GX_CHUNK:(c + 1) * GX_CHUNK] = gx.reshape(GX_CHUNK, BB, 4 * HID)

    whha = whha_ref[...].astype(bf16)

    def sig(x):
        return 0.5 * jnp.tanh(0.5 * x) + 0.5

    def act(gates, c):
        i_t = sig(gates[:, 0 * HID:1 * HID])
        f_t = sig(gates[:, 1 * HID:2 * HID])
        g_t = jnp.tanh(gates[:, 2 * HID:3 * HID])
        o_t = sig(gates[:, 3 * HID:4 * HID])
        c_new = f_t * c + i_t * g_t
        h_new = o_t * jnp.tanh(c_new)
        return h_new, c_new

    ones_h = jnp.ones((HB, 1), dtype=bf16)
    zero = jnp.zeros((HB, HID), dtype=f32)
    ha, ca, hb, cb = zero, zero, zero, zero
    for t in range(T):
        gx = gx_scr[t].astype(f32)
        haa = jnp.concatenate([ha.astype(bf16), ones_h], axis=1)
        hba = jnp.concatenate([hb.astype(bf16), ones_h], axis=1)
        ga = gx[0:HB] + jnp.dot(haa, whha, preferred_element_type=f32)
        gb = gx[HB:BB] + jnp.dot(hba, whha, preferred_element_type=f32)
        ha, ca = act(ga, ca)
        hb, cb = act(gb, cb)
    h = jnp.concatenate([ha, hb], axis=0)

    hc = jnp.maximum(jnp.dot(h, cw1_ref[...], preferred_element_type=f32)
                     + cb1_ref[...], 0.0)
    coords_ref[...] = jnp.dot(hc, cw2_ref[...], preferred_element_type=f32) + cb2_ref[...]
    hl = jnp.maximum(jnp.dot(h, lw1_ref[...], preferred_element_type=f32)
                     + lb1_ref[...], 0.0)
    labels_ref[...] = jnp.dot(hl, lw2_ref[...], preferred_element_type=f32) + lb2_ref[...]


def _full(shape):
    return pl.BlockSpec(shape, lambda i: (0,) * len(shape))


@functools.partial(jax.jit, static_argnames=("interpret",))
def _run(x, W1a, W2a, W_ih, Whha,
         coord_W1, coord_b1, coord_W2, coord_b2,
         lab_W1, lab_b1, lab_W2, lab_b2, interpret=False):
    n_tiles = B // BB
    out_shapes = (
        jax.ShapeDtypeStruct((B, 3 * STEPS), jnp.float32),
        jax.ShapeDtypeStruct((B, STEPS), jnp.float32),
    )
    return pl.pallas_call(
        _fused_kernel,
        grid=(n_tiles,),
        in_specs=[
            pl.BlockSpec((T, BB, D_INA), lambda i: (0, i, 0)),
            _full((D_INA, ENC)),
            _full((ENC + 1, ENC)),
            _full((ENC, 4 * HID)),
            _full((HID + 1, 4 * HID)),
            _full((HID, HID)), _full((1, HID)),
            _full((HID, 3 * STEPS)), _full((1, 3 * STEPS)),
            _full((HID, HID // 2)), _full((1, HID // 2)),
            _full((HID // 2, STEPS)), _full((1, STEPS)),
        ],
        out_specs=(
            pl.BlockSpec((BB, 3 * STEPS), lambda i: (i, 0)),
            pl.BlockSpec((BB, STEPS), lambda i: (i, 0)),
        ),
        out_shape=out_shapes,
        scratch_shapes=[pltpu.VMEM((T, BB, 4 * HID), jnp.bfloat16)],
        compiler_params=pltpu.CompilerParams(
            dimension_semantics=("parallel",),
        ),
        interpret=interpret,
    )(x, W1a, W2a, W_ih, Whha,
      coord_W1, coord_b1, coord_W2, coord_b2,
      lab_W1, lab_b1, lab_W2, lab_b2)


def kernel(obs_l, obs_c, obs_m, enc_W1, enc_b1, enc_W2, enc_b2,
           W_ih, W_hh, b_ih, b_hh,
           coord_W1, coord_b1, coord_W2, coord_b2,
           lab_W1, lab_b1, lab_W2, lab_b2):
    ones = jnp.ones((B, T, 1), dtype=obs_l.dtype)
    x = jnp.concatenate([obs_l, obs_c, obs_m, ones], axis=-1)
    x = jnp.swapaxes(x, 0, 1).astype(jnp.bfloat16)
    W1a = jnp.concatenate([enc_W1, enc_b1[None, :]], axis=0)
    W2a = jnp.concatenate([enc_W2, enc_b2[None, :]], axis=0)
    Whha = jnp.concatenate([W_hh, (b_ih + b_hh)[None, :]], axis=0)
    return _run(x, W1a, W2a, W_ih, Whha,
                coord_W1, coord_b1.reshape(1, HID), coord_W2, coord_b2.reshape(1, 3 * STEPS),
                lab_W1, lab_b1.reshape(1, HID // 2), lab_W2, lab_b2.reshape(1, STEPS))
